# baseline (speedup 1.0000x reference)
import functools

import jax
import jax.numpy as jnp
from jax import lax
from jax.experimental import pallas as pl
from jax.experimental.pallas import tpu as pltpu
from jax.experimental.pallas import tpu_sc as plsc

N_NODES = 10000
DIM = 128
HD = DIM // 2
N_EDGES = 320000
N_REL = 964

NC = 2
NS = 16
NW = NC * NS
L = 16

CH = N_EDGES // NW
TPAD = 976
NV = TPAD // L
B = 128
NBF = CH // B
TB = CH - NBF * B
NBW = NBF + 1

_mesh = plsc.VectorSubcoreMesh(core_axis_name="c", subcore_axis_name="s")


def _wid():
    return lax.axis_index("c") * NS + lax.axis_index("s")


@functools.partial(
    pl.kernel,
    out_type=jax.ShapeDtypeStruct((NW, TPAD), jnp.int32),
    mesh=_mesh,
    compiler_params=pltpu.CompilerParams(use_tc_tiling_on_sc=False,
                                         needs_layout_passes=False),
    scratch_types=[
        pltpu.VMEM((CH,), jnp.int32),
        pltpu.VMEM((TPAD,), jnp.int32),
    ],
)
def _hist_kernel(et_hbm, hist_hbm, et_v, h1d):
    wid = _wid()
    pltpu.sync_copy(et_hbm.at[pl.ds(wid * CH, CH)], et_v)

    zero16 = jnp.zeros((L,), jnp.int32)

    def zero_body(j, _):
        h1d[pl.ds(j * L, L)] = zero16
        return 0

    lax.fori_loop(0, NV, zero_body, 0)

    ones = jnp.ones((L,), jnp.int32)

    def hist_body(g, _):
        tv = et_v[pl.ds(g * L, L)]
        plsc.addupdate_scatter(h1d, [tv], ones)
        return 0

    lax.fori_loop(0, CH // L, hist_body, 0)
    pltpu.sync_copy(h1d, hist_hbm.at[wid])


@functools.partial(
    pl.kernel,
    out_type=jax.ShapeDtypeStruct((N_EDGES,), jnp.float32),
    mesh=_mesh,
    compiler_params=pltpu.CompilerParams(use_tc_tiling_on_sc=False,
                                         needs_layout_passes=False),
    scratch_types=[
        pltpu.VMEM((N_REL, HD + 1), jnp.int32),
        pltpu.VMEM((B, HD + 1), jnp.int32),
        pltpu.VMEM((B, L), jnp.float32),
        pltpu.VMEM((4, TPAD), jnp.int32),
        pltpu.VMEM((TPAD,), jnp.int32),
        pltpu.VMEM((TPAD,), jnp.int32),
        pltpu.VMEM((2, 3 * B), jnp.int32),
        pltpu.VMEM((2, B), jnp.int32),
        pltpu.VMEM((2, B, HD), jnp.int32),
        pltpu.VMEM((2, B, HD), jnp.int32),
        pltpu.VMEM((2, B), jnp.int32),
        pltpu.VMEM((2, B), jnp.float32),
        pltpu.VMEM((TB,), jnp.int32),
        pltpu.VMEM((TB,), jnp.float32),
        pltpu.VMEM((3 * L,), jnp.int32),
        [pltpu.SemaphoreType.DMA] * 2,
        [pltpu.SemaphoreType.DMA] * 2,
        [pltpu.SemaphoreType.DMA] * 2,
        [pltpu.SemaphoreType.DMA] * 2,
    ],
)
def _main_kernel(x_hbm, ids_hbm, r_hbm, hist_hbm, out_hbm,
                 rels_v, relx, part_v, row4_v, tot_v, base_v,
                 idb, tvb, xl2, xr2, pos2, sc2, pos_t, sc_t, tbuf,
                 glsems, grsems, isems, ssems):
    wid = _wid()
    pltpu.sync_copy(r_hbm, rels_v)
    chunk0 = wid * CH
    wid_v = jnp.zeros((L,), jnp.int32) + wid
    zero16 = jnp.zeros((L,), jnp.int32)

    def zero_body(j, _):
        tot_v[pl.ds(j * L, L)] = zero16
        base_v[pl.ds(j * L, L)] = zero16
        return 0

    lax.fori_loop(0, NV, zero_body, 0)

    for piece in range(NW // 4):
        pltpu.sync_copy(hist_hbm.at[pl.ds(piece * 4, 4)], row4_v)

        def pc_body(j, _):
            js = pl.ds(j * L, L)
            t = tot_v[js]
            p = base_v[js]
            for c4 in range(4):
                c = piece * 4 + c4
                v = row4_v[c4, js]
                t = t + v
                p = jnp.where(jnp.full((L,), c, jnp.int32) < wid_v, p + v, p)
            tot_v[js] = t
            base_v[js] = p
            return 0

        lax.fori_loop(0, NV, pc_body, 0)

    def scan_body(j, carry):
        js = pl.ds(j * L, L)
        tot = tot_v[js]
        inc = plsc.cumsum(tot)
        base_v[js] = base_v[js] + (inc - tot) + carry
        return carry + jnp.sum(tot)

    lax.fori_loop(0, NV, scan_body, jnp.int32(0))

    tbuf[pl.ds(0, L)] = jnp.full((L,), -1, jnp.int32)
    tbuf[pl.ds(2 * L, L)] = jnp.full((L,), -2, jnp.int32)
    lanes = lax.iota(jnp.int32, L)
    ones = jnp.ones((L,), jnp.int32)

    def id_start(b, s):
        pltpu.async_copy(ids_hbm.at[wid * NBW + b], idb.at[s], isems[s])

    def id_wait(s):
        pltpu.make_async_copy(ids_hbm.at[0], idb.at[s], isems[s]).wait()

    def g_start(s):
        pltpu.async_copy(x_hbm.at[idb.at[s, pl.ds(0, B)]], xl2.at[s],
                         glsems[s])
        pltpu.async_copy(x_hbm.at[idb.at[s, pl.ds(B, B)]], xr2.at[s],
                         grsems[s])

    def g_wait(s):
        pltpu.make_async_copy(x_hbm.at[idb.at[s, pl.ds(0, B)]], xl2.at[s],
                              glsems[s]).wait()
        pltpu.make_async_copy(x_hbm.at[idb.at[s, pl.ds(B, B)]], xr2.at[s],
                              grsems[s]).wait()

    def s_drain(s):
        pltpu.make_async_copy(sc2.at[s], out_hbm.at[pos2.at[s]],
                              ssems[s]).wait()

    fifteen = jnp.full((L,), L - 1, jnp.int32)

    def tv_copy(s):
        for g in range(B // L):
            tvb[s, pl.ds(g * L, L)] = idb[s, pl.ds(2 * B + g * L, L)]

    def pos_group(tv, pos_ref, sl):
        tbuf[pl.ds(L, L)] = tv
        rank = zero16
        for k in range(1, L):
            shm = tbuf[pl.ds(L - k, L)]
            rank = rank + jnp.where(shm == tv, ones, zero16)
        gb = plsc.load_gather(base_v, [tv])
        pos_ref[sl] = gb + rank
        plsc.addupdate_scatter(base_v, [tv], ones)

    def rel_expand(tv, g):
        rows = lanes + (g * L)

        @plsc.parallel_loop(0, HD, unroll=2)
        def _(c):
            colc = zero16 + c
            rw = plsc.load_gather(rels_v, [tv, colc])
            plsc.store_scatter(relx, [rows, colc], rw)

    def score_rows(s, n):
        @plsc.parallel_loop(0, n, unroll=2)
        def _(i):
            acc_a = jnp.zeros((L,), jnp.float32)
            acc_b = jnp.zeros((L,), jnp.float32)
            for j in range(DIM // (2 * L)):
                sl = pl.ds(j * L, L)
                lb = plsc.bitcast(xl2[s, i, sl], jnp.bfloat16)
                xb = plsc.bitcast(xr2[s, i, sl], jnp.bfloat16)
                rb = plsc.bitcast(relx[i, sl], jnp.bfloat16)
                prod = lb * xb
                pa, pb = plsc.unpack(prod, format=plsc.PackFormat.INTERLEAVED)
                ra, rb2 = plsc.unpack(rb, format=plsc.PackFormat.INTERLEAVED)
                acc_a = acc_a + pa * ra
                acc_b = acc_b + pb * rb2
            part_v[i] = plsc.cumsum(acc_a + acc_b)

    def post(s):
        for g in range(B // L):
            tv = tvb[s, pl.ds(g * L, L)]
            pos_group(tv, pos2.at[s], pl.ds(g * L, L))
            rel_expand(tv, g)
        score_rows(s, B)
        for g in range(B // L):
            eids = lanes + (g * L)
            tot = plsc.load_gather(part_v, [eids, fifteen])
            sc2[s, pl.ds(g * L, L)] = 1.0 / (1.0 + jnp.exp(-tot))
        pltpu.async_copy(sc2.at[s], out_hbm.at[pos2.at[s]], ssems[s])

    pltpu.sync_copy(ids_hbm.at[wid * NBW + 0], idb.at[0])
    g_start(0)
    id_start(1, 1)
    NH = NBF // 2

    def body(h, _):
        b0 = 2 * h

        @pl.when(h > 0)
        def _():
            s_drain(0)
            s_drain(1)

        id_wait(1)
        g_start(1)
        g_wait(0)
        tv_copy(0)
        id_start(b0 + 2, 0)
        post(0)
        id_wait(0)
        g_start(0)
        g_wait(1)
        tv_copy(1)

        @pl.when(h < NH - 1)
        def _():
            id_start(b0 + 3, 1)

        post(1)
        return 0

    lax.fori_loop(0, NH, body, 0)
    s_drain(0)
    s_drain(1)
    g_wait(0)
    tvt = idb[0, pl.ds(2 * B, L)]
    pos_group(tvt, pos_t, pl.ds(0, L))
    rel_expand(tvt, 0)
    score_rows(0, TB)
    tott = plsc.load_gather(part_v, [lanes, fifteen])
    sc_t[pl.ds(0, L)] = 1.0 / (1.0 + jnp.exp(-tott))
    pltpu.async_copy(sc_t, out_hbm.at[pos_t], ssems[0]).wait()


def _pack_rows(a):
    b = a.astype(jnp.bfloat16)
    return jax.lax.bitcast_convert_type(
        b.reshape(a.shape[0], a.shape[1] // 2, 2), jnp.int32)


def kernel(x, edge_index, edge_type, R):
    left = edge_index[0]
    right = edge_index[1]
    hist = _hist_kernel(edge_type)
    r_pad = jnp.pad(_pack_rows(R), ((0, 0), (0, 1)))

    def blocked(a):
        ap = jnp.pad(a.reshape(NW, CH), ((0, 0), (0, NBW * B - CH)))
        return ap.reshape(NW, NBW, B)

    ids_blk = jnp.concatenate(
        [blocked(left), blocked(right), blocked(edge_type)],
        axis=-1).reshape(NW * NBW, 3 * B)
    return _main_kernel(_pack_rows(x), ids_blk, r_pad, hist)


# speedup vs baseline: 1.2380x; 1.2380x over previous
import functools

import jax
import jax.numpy as jnp
from jax import lax
from jax.experimental import pallas as pl
from jax.experimental.pallas import tpu as pltpu
from jax.experimental.pallas import tpu_sc as plsc

N_NODES = 10000
DIM = 128
N_EDGES = 320000
N_REL = 964

NC = 2
NS = 16
NW = NC * NS
L = 16

CH = N_EDGES // NW
TPAD = 976
NV = TPAD // L
B = 80
NB = CH // B

_mesh = plsc.VectorSubcoreMesh(core_axis_name="c", subcore_axis_name="s")


def _wid():
    return lax.axis_index("c") * NS + lax.axis_index("s")


@functools.partial(
    pl.kernel,
    out_type=jax.ShapeDtypeStruct((NW, TPAD), jnp.int32),
    mesh=_mesh,
    compiler_params=pltpu.CompilerParams(use_tc_tiling_on_sc=False,
                                         needs_layout_passes=False),
    scratch_types=[
        pltpu.VMEM((CH,), jnp.int32),
        pltpu.VMEM((TPAD,), jnp.int32),
    ],
)
def _hist_kernel(et_hbm, hist_hbm, et_v, h1d):
    wid = _wid()
    pltpu.sync_copy(et_hbm.at[pl.ds(wid * CH, CH)], et_v)

    zero16 = jnp.zeros((L,), jnp.int32)

    def zero_body(j, _):
        h1d[pl.ds(j * L, L)] = zero16
        return 0

    lax.fori_loop(0, NV, zero_body, 0)

    ones = jnp.ones((L,), jnp.int32)

    def hist_body(g, _):
        tv = et_v[pl.ds(g * L, L)]
        plsc.addupdate_scatter(h1d, [tv], ones)
        return 0

    lax.fori_loop(0, CH // L, hist_body, 0)
    pltpu.sync_copy(h1d, hist_hbm.at[wid])


@functools.partial(
    pl.kernel,
    out_type=jax.ShapeDtypeStruct((N_EDGES,), jnp.float32),
    mesh=_mesh,
    compiler_params=pltpu.CompilerParams(use_tc_tiling_on_sc=False,
                                         needs_layout_passes=False),
    scratch_types=[
        pltpu.VMEM((NW, TPAD), jnp.int32),
        pltpu.VMEM((TPAD,), jnp.int32),
        pltpu.VMEM((CH,), jnp.int32),
        pltpu.VMEM((CH,), jnp.int32),
        pltpu.VMEM((CH,), jnp.int32),
        pltpu.VMEM((2, B, DIM), jnp.bfloat16),
        pltpu.VMEM((2, B, DIM), jnp.bfloat16),
        pltpu.VMEM((2, B, DIM), jnp.bfloat16),
        pltpu.VMEM((2, B), jnp.int32),
        pltpu.VMEM((2, B), jnp.float32),
        pltpu.VMEM((B, L), jnp.float32),
        pltpu.VMEM((3 * L,), jnp.int32),
        pltpu.SemaphoreType.DMA,
        pltpu.SemaphoreType.DMA,
        pltpu.SemaphoreType.DMA,
        pltpu.SemaphoreType.DMA,
        pltpu.SemaphoreType.DMA,
        pltpu.SemaphoreType.DMA,
        pltpu.SemaphoreType.DMA,
        pltpu.SemaphoreType.DMA,
    ],
)
def _main_kernel(x_hbm, left_hbm, right_hbm, et_hbm, r_hbm, hist_hbm, out_hbm,
                 hist_v, base_v, lid_all, rid_all, tid_all, xl2, xr2, rel2,
                 pos2, sc2, part_v, tbuf,
                 gsem0, gsem1, gsem2, gsem3, gsem4, gsem5, ssem0, ssem1):
    wid = _wid()
    pltpu.sync_copy(hist_hbm, hist_v)
    chunk0 = wid * CH
    pltpu.sync_copy(left_hbm.at[pl.ds(chunk0, CH)], lid_all)
    pltpu.sync_copy(right_hbm.at[pl.ds(chunk0, CH)], rid_all)
    pltpu.sync_copy(et_hbm.at[pl.ds(chunk0, CH)], tid_all)
    wid_v = jnp.zeros((L,), jnp.int32) + wid

    def off_body(j, carry):
        tot = hist_v[0, pl.ds(j * L, L)]
        prior = jnp.where(jnp.zeros((L,), jnp.int32) < wid_v, tot,
                          jnp.zeros((L,), jnp.int32))
        for c in range(1, NW):
            v = hist_v[c, pl.ds(j * L, L)]
            tot = tot + v
            prior = jnp.where(jnp.full((L,), c, jnp.int32) < wid_v,
                              prior + v, prior)
        inc = plsc.cumsum(tot)
        base_v[pl.ds(j * L, L)] = (inc - tot) + prior + carry
        return carry + jnp.sum(tot)

    lax.fori_loop(0, NV, off_body, jnp.int32(0))

    tbuf[pl.ds(0, L)] = jnp.full((L,), -1, jnp.int32)
    tbuf[pl.ds(2 * L, L)] = jnp.full((L,), -2, jnp.int32)
    lanes = lax.iota(jnp.int32, L)
    ones = jnp.ones((L,), jnp.int32)
    zeros = jnp.zeros((L,), jnp.int32)
    fifteen = jnp.full((L,), L - 1, jnp.int32)

    gsems = ((gsem0, gsem1, gsem2), (gsem3, gsem4, gsem5))
    ssems = (ssem0, ssem1)

    def g_start(b, s):
        i0 = pl.ds(b * B, B)
        pltpu.async_copy(x_hbm.at[lid_all.at[i0]], xl2.at[s], gsems[s][0])
        pltpu.async_copy(x_hbm.at[rid_all.at[i0]], xr2.at[s], gsems[s][1])
        pltpu.async_copy(r_hbm.at[tid_all.at[i0]], rel2.at[s], gsems[s][2])

    def g_wait(s):
        i0 = pl.ds(0, B)
        pltpu.make_async_copy(x_hbm.at[lid_all.at[i0]], xl2.at[s],
                              gsems[s][0]).wait()
        pltpu.make_async_copy(x_hbm.at[rid_all.at[i0]], xr2.at[s],
                              gsems[s][1]).wait()
        pltpu.make_async_copy(r_hbm.at[tid_all.at[i0]], rel2.at[s],
                              gsems[s][2]).wait()

    def s_drain(s):
        pltpu.make_async_copy(sc2.at[s], out_hbm.at[pos2.at[s]],
                              ssems[s]).wait()

    def compute(b, s):
        for g in range(B // L):
            tv = tid_all[pl.ds(b * B + g * L, L)]
            tbuf[pl.ds(L, L)] = tv
            rank = zeros
            for k in range(1, L):
                shm = tbuf[pl.ds(L - k, L)]
                rank = rank + jnp.where(shm == tv, ones, zeros)
            gb = plsc.load_gather(base_v, [tv])
            pos2[s, pl.ds(g * L, L)] = gb + rank
            plsc.addupdate_scatter(base_v, [tv], ones)

        @plsc.parallel_loop(0, B, unroll=2)
        def _(i):
            acc = None
            for j in range(DIM // (2 * L)):
                sl = pl.ds(j * 2 * L, 2 * L)
                la, lb = plsc.unpack(xl2[s, i, sl],
                                     format=plsc.PackFormat.INTERLEAVED)
                ra, rb = plsc.unpack(rel2[s, i, sl],
                                     format=plsc.PackFormat.INTERLEAVED)
                xa, xb = plsc.unpack(xr2[s, i, sl],
                                     format=plsc.PackFormat.INTERLEAVED)
                term = la * ra * xa + lb * rb * xb
                acc = term if acc is None else acc + term
            part_v[i] = plsc.cumsum(acc)

        for g in range(B // L):
            eids = lanes + (g * L)
            tot = plsc.load_gather(part_v, [eids, fifteen])
            sc2[s, pl.ds(g * L, L)] = 1.0 / (1.0 + jnp.exp(-tot))

        pltpu.async_copy(sc2.at[s], out_hbm.at[pos2.at[s]], ssems[s])

    g_start(0, 0)

    def body(h, _):
        b0 = 2 * h

        @pl.when(h > 0)
        def _():
            s_drain(0)
            s_drain(1)

        g_start(b0 + 1, 1)
        g_wait(0)
        compute(b0, 0)
        g_start(b0 + 2, 0)
        g_wait(1)
        compute(b0 + 1, 1)
        return 0

    lax.fori_loop(0, (NB - 1) // 2, body, 0)
    s_drain(0)
    s_drain(1)
    g_wait(0)
    compute(NB - 1, 0)
    s_drain(0)


def kernel(x, edge_index, edge_type, R):
    left = edge_index[0]
    right = edge_index[1]
    hist = _hist_kernel(edge_type)
    return _main_kernel(x.astype(jnp.bfloat16), left, right, edge_type,
                        R.astype(jnp.bfloat16), hist)
